# baseline (speedup 1.0000x reference)
import functools

import jax
import jax.numpy as jnp
from jax.experimental import pallas as pl
from jax.experimental.pallas import tpu as pltpu


def _gcn_body(y_ref, w1_ref, b1_ref, w2_ref, b2_ref, m1_ref, c1_ref,
              m2_ref, c2_ref, out_ref, r_scr, *, n_batch):
    b = pl.program_id(0)
    n = y_ref.shape[1]
    nh = w1_ref.shape[0]
    b1c = b1_ref[...].reshape(nh, 1)
    b2c = b2_ref[...].reshape(nh, 1)
    c1r = c1_ref[...].reshape(1, -1)
    c2r = c2_ref[...].reshape(1, -1)

    tile = 128
    row_i = jax.lax.broadcasted_iota(jnp.int32, (tile, tile), 0)
    col_i = jax.lax.broadcasted_iota(jnp.int32, (tile, tile), 1)
    mask = row_i == col_i
    diag = jnp.concatenate(
        [jnp.sum(jnp.where(mask,
                           y_ref[0, t * tile:(t + 1) * tile,
                                 t * tile:(t + 1) * tile], 0.0),
                 axis=0, keepdims=True)
         for t in range(n // tile)], axis=1)
    loop_w = jnp.where(diag == 0.0, 1.0, 0.0)

    parts = []
    bf_slabs = []
    for c in range(n // tile):
        v = y_ref[0, c * tile:(c + 1) * tile, :]
        parts.append(jnp.sum(v, axis=0, keepdims=True))
        bf_slabs.append(v.astype(jnp.bfloat16))
    colsum = functools.reduce(jnp.add, parts)
    a_bf = jnp.concatenate(bf_slabs, axis=0)

    deg = colsum + loop_w
    dinv = jnp.where(deg > 0.0, jax.lax.rsqrt(jnp.where(deg > 0.0, deg, 1.0)),
                     0.0)

    t1 = jnp.dot(dinv.astype(jnp.bfloat16), a_bf,
                 preferred_element_type=jnp.float32)
    s = dinv * (t1 + dinv * loop_w)
    x1t = jnp.maximum(w1_ref[...] * s + b1c, 0.0)

    h2t = jnp.dot(w2_ref[...], x1t,
                  preferred_element_type=jnp.float32)
    gt = dinv * h2t
    zt = jnp.dot(gt.astype(jnp.bfloat16), a_bf,
                 preferred_element_type=jnp.float32)
    out2t = dinv * (zt + loop_w * gt) + b2c
    r_scr[pl.ds(b, 1), :] = jnp.max(out2t, axis=0, keepdims=True)

    @pl.when(b == n_batch - 1)
    def _():
        rr = r_scr[...]
        o1 = jax.lax.dot_general(
            rr, m1_ref[...], (((1,), (1,)), ((), ())),
            preferred_element_type=jnp.float32) + c1r
        o2 = jax.lax.dot_general(
            o1, m2_ref[...], (((1,), (1,)), ((), ())),
            preferred_element_type=jnp.float32) + c2r
        out_ref[...] = o2


def kernel(y, W1, b1, W2, b2, M1, c1, M2, c2):
    B, N = y.shape[0], y.shape[1]

    vmem = pl.BlockSpec(memory_space=pltpu.MemorySpace.VMEM)
    return pl.pallas_call(
        functools.partial(_gcn_body, n_batch=B),
        grid=(B,),
        in_specs=[
            pl.BlockSpec((1, N, N), lambda b: (b, 0, 0)),
            vmem, vmem, vmem, vmem, vmem, vmem, vmem, vmem,
        ],
        out_specs=pl.BlockSpec((B, M2.shape[0]), lambda b: (0, 0)),
        out_shape=jax.ShapeDtypeStruct((B, M2.shape[0]), jnp.float32),
        scratch_shapes=[pltpu.VMEM((B, N), jnp.float32)],
    )(y, W1, b1, W2, b2, M1, c1, M2, c2)
